# Initial kernel scaffold; baseline (speedup 1.0000x reference)
import jax
import jax.numpy as jnp
from jax.experimental import pallas as pl


def kernel(gender_idx, age_idx, occupation_idx, W_gender, W_age, W_occupation):
    raise NotImplementedError("write your pallas kernel here")


# speedup vs baseline: 1.1061x; 1.1061x over previous
import functools

import jax
import jax.numpy as jnp
from jax import lax
from jax.experimental import pallas as pl
from jax.experimental.pallas import tpu as pltpu
from jax.experimental.pallas import tpu_sc as plsc

B = 16384
D = 32


def kernel(gender_idx, age_idx, occupation_idx, W_gender, W_age, W_occupation):
    info = plsc.get_sparse_core_info()
    nw = info.num_cores * info.num_subcores
    b_per_w = B // nw
    mesh = plsc.VectorSubcoreMesh(core_axis_name="c", subcore_axis_name="s")

    @functools.partial(
        pl.kernel,
        mesh=mesh,
        out_type=jax.ShapeDtypeStruct((B, 3 * D), jnp.float32),
        compiler_params=pltpu.CompilerParams(use_tc_tiling_on_sc=False),
        scratch_types=[
            pltpu.VMEM((b_per_w,), jnp.int32),
            pltpu.VMEM((b_per_w,), jnp.int32),
            pltpu.VMEM((b_per_w,), jnp.int32),
            pltpu.VMEM((b_per_w, D), jnp.float32),
            pltpu.VMEM((b_per_w, D), jnp.float32),
            pltpu.VMEM((b_per_w, D), jnp.float32),
            pltpu.SemaphoreType.DMA,
            pltpu.SemaphoreType.DMA,
            pltpu.SemaphoreType.DMA,
        ],
    )
    def emb(g_hbm, a_hbm, o_hbm, wg_hbm, wa_hbm, wo_hbm, out_hbm,
            gi_v, ai_v, oi_v, gr_v, ar_v, or_v, sg, sa, so):
        wid = lax.axis_index("s") * info.num_cores + lax.axis_index("c")
        base = wid * b_per_w
        pltpu.sync_copy(g_hbm.at[pl.ds(base, b_per_w)], gi_v)
        pltpu.sync_copy(a_hbm.at[pl.ds(base, b_per_w)], ai_v)
        pltpu.sync_copy(o_hbm.at[pl.ds(base, b_per_w)], oi_v)
        cg = pltpu.async_copy(wg_hbm.at[gi_v], gr_v, sg)
        ca = pltpu.async_copy(wa_hbm.at[ai_v], ar_v, sa)
        co = pltpu.async_copy(wo_hbm.at[oi_v], or_v, so)
        cg.wait()
        pltpu.sync_copy(gr_v, out_hbm.at[pl.ds(base, b_per_w), pl.ds(0, D)])
        ca.wait()
        pltpu.sync_copy(ar_v, out_hbm.at[pl.ds(base, b_per_w), pl.ds(D, D)])
        co.wait()
        pltpu.sync_copy(or_v, out_hbm.at[pl.ds(base, b_per_w), pl.ds(2 * D, D)])

    return emb(gender_idx, age_idx, occupation_idx,
               W_gender, W_age, W_occupation)


# speedup vs baseline: 2.2562x; 2.0398x over previous
import functools

import jax
import jax.numpy as jnp
from jax import lax
from jax.experimental import pallas as pl
from jax.experimental.pallas import tpu as pltpu
from jax.experimental.pallas import tpu_sc as plsc

B = 16384
D = 32
L = 16


def kernel(gender_idx, age_idx, occupation_idx, W_gender, W_age, W_occupation):
    info = plsc.get_sparse_core_info()
    nw = info.num_cores * info.num_subcores
    b_per_w = B // nw
    n_groups = b_per_w // L
    mesh = plsc.VectorSubcoreMesh(core_axis_name="c", subcore_axis_name="s")

    @functools.partial(
        pl.kernel,
        mesh=mesh,
        out_type=jax.ShapeDtypeStruct((B, 3 * D), jnp.float32),
        compiler_params=pltpu.CompilerParams(use_tc_tiling_on_sc=False,
                                             needs_layout_passes=False),
        scratch_types=[
            pltpu.VMEM((b_per_w,), jnp.int32),
            pltpu.VMEM((b_per_w,), jnp.int32),
            pltpu.VMEM((b_per_w,), jnp.int32),
            pltpu.VMEM((2, D), jnp.float32),
            pltpu.VMEM((7, D), jnp.float32),
            pltpu.VMEM((21, D), jnp.float32),
            pltpu.VMEM((b_per_w, 3 * D), jnp.float32),
        ],
    )
    def emb(g_hbm, a_hbm, o_hbm, wg_hbm, wa_hbm, wo_hbm, out_hbm,
            gi_v, ai_v, oi_v, tg_v, ta_v, to_v, stage_v):
        wid = lax.axis_index("s") * info.num_cores + lax.axis_index("c")
        base = wid * b_per_w
        pltpu.sync_copy(g_hbm.at[pl.ds(base, b_per_w)], gi_v)
        pltpu.sync_copy(a_hbm.at[pl.ds(base, b_per_w)], ai_v)
        pltpu.sync_copy(o_hbm.at[pl.ds(base, b_per_w)], oi_v)
        pltpu.sync_copy(wg_hbm, tg_v)
        pltpu.sync_copy(wa_hbm, ta_v)
        pltpu.sync_copy(wo_hbm, to_v)

        lanes = lax.iota(jnp.int32, L)

        def body(i, carry):
            rows = (gi_v[pl.ds(i * L, L)],
                    ai_v[pl.ds(i * L, L)],
                    oi_v[pl.ds(i * L, L)])
            rpos = lanes + i * L
            for f, tv in enumerate((tg_v, ta_v, to_v)):
                for cc in range(D):
                    col = jnp.full((L,), cc, jnp.int32)
                    val = plsc.load_gather(tv, [rows[f], col])
                    out_col = jnp.full((L,), f * D + cc, jnp.int32)
                    plsc.store_scatter(stage_v, [rpos, out_col], val)
            return carry

        lax.fori_loop(0, n_groups, body, 0)
        pltpu.sync_copy(stage_v, out_hbm.at[pl.ds(base, b_per_w)])

    return emb(gender_idx, age_idx, occupation_idx,
               W_gender, W_age, W_occupation)


# speedup vs baseline: 2.4594x; 1.0901x over previous
import functools

import jax
import jax.numpy as jnp
from jax import lax
from jax.experimental import pallas as pl
from jax.experimental.pallas import tpu as pltpu
from jax.experimental.pallas import tpu_sc as plsc

B = 16384
D = 32
L = 16


def kernel(gender_idx, age_idx, occupation_idx, W_gender, W_age, W_occupation):
    info = plsc.get_sparse_core_info()
    nw = info.num_cores * info.num_subcores
    b_per_w = B // nw
    n_groups = b_per_w // L
    mesh = plsc.VectorSubcoreMesh(core_axis_name="c", subcore_axis_name="s")

    @functools.partial(
        pl.kernel,
        mesh=mesh,
        out_type=jax.ShapeDtypeStruct((B, 3 * D), jnp.float32),
        compiler_params=pltpu.CompilerParams(use_tc_tiling_on_sc=False,
                                             needs_layout_passes=False),
        scratch_types=[
            pltpu.VMEM((b_per_w,), jnp.int32),
            pltpu.VMEM((b_per_w,), jnp.int32),
            pltpu.VMEM((b_per_w,), jnp.int32),
            pltpu.VMEM((2, D), jnp.float32),
            pltpu.VMEM((7, D), jnp.float32),
            pltpu.VMEM((21, D), jnp.float32),
            pltpu.VMEM((b_per_w, 3 * D), jnp.float32),
        ],
    )
    def emb(g_hbm, a_hbm, o_hbm, wg_hbm, wa_hbm, wo_hbm, out_hbm,
            gi_v, ai_v, oi_v, tg_v, ta_v, to_v, stage_v):
        wid = lax.axis_index("s") * info.num_cores + lax.axis_index("c")
        base = wid * b_per_w
        pltpu.sync_copy(g_hbm.at[pl.ds(base, b_per_w)], gi_v)
        pltpu.sync_copy(a_hbm.at[pl.ds(base, b_per_w)], ai_v)
        pltpu.sync_copy(o_hbm.at[pl.ds(base, b_per_w)], oi_v)
        pltpu.sync_copy(wg_hbm, tg_v)
        pltpu.sync_copy(wa_hbm, ta_v)
        pltpu.sync_copy(wo_hbm, to_v)

        lanes = lax.iota(jnp.int32, L)

        @plsc.parallel_loop(0, n_groups, step=1, unroll=2)
        def body(i):
            rows = (gi_v[pl.ds(i * L, L)],
                    ai_v[pl.ds(i * L, L)],
                    oi_v[pl.ds(i * L, L)])
            rpos = lanes + i * L
            for f, tv in enumerate((tg_v, ta_v, to_v)):
                for cc in range(D):
                    col = jnp.full((L,), cc, jnp.int32)
                    val = plsc.load_gather(tv, [rows[f], col])
                    out_col = jnp.full((L,), f * D + cc, jnp.int32)
                    plsc.store_scatter(stage_v, [rpos, out_col], val)
        pltpu.sync_copy(stage_v, out_hbm.at[pl.ds(base, b_per_w)])

    return emb(gender_idx, age_idx, occupation_idx,
               W_gender, W_age, W_occupation)


# speedup vs baseline: 2.5139x; 1.0222x over previous
import functools

import jax
import jax.numpy as jnp
from jax import lax
from jax.experimental import pallas as pl
from jax.experimental.pallas import tpu as pltpu
from jax.experimental.pallas import tpu_sc as plsc

B = 16384
D = 32
L = 16
OUT_D = 3 * D


def kernel(gender_idx, age_idx, occupation_idx, W_gender, W_age, W_occupation):
    info = plsc.get_sparse_core_info()
    nw = info.num_cores * info.num_subcores
    b_per_w = B // nw
    n_groups = b_per_w // L
    mesh = plsc.VectorSubcoreMesh(core_axis_name="c", subcore_axis_name="s")

    @functools.partial(
        pl.kernel,
        mesh=mesh,
        out_type=jax.ShapeDtypeStruct((B * OUT_D,), jnp.float32),
        compiler_params=pltpu.CompilerParams(use_tc_tiling_on_sc=False,
                                             needs_layout_passes=False),
        scratch_types=[
            pltpu.VMEM((b_per_w,), jnp.int32),
            pltpu.VMEM((b_per_w,), jnp.int32),
            pltpu.VMEM((b_per_w,), jnp.int32),
            pltpu.VMEM((2 * D,), jnp.float32),
            pltpu.VMEM((7 * D,), jnp.float32),
            pltpu.VMEM((21 * D,), jnp.float32),
            pltpu.VMEM((b_per_w * OUT_D,), jnp.float32),
        ],
    )
    def emb(g_hbm, a_hbm, o_hbm, wg_hbm, wa_hbm, wo_hbm, out_hbm,
            gi_v, ai_v, oi_v, tg_v, ta_v, to_v, stage_v):
        wid = lax.axis_index("s") * info.num_cores + lax.axis_index("c")
        base = wid * b_per_w
        pltpu.sync_copy(g_hbm.at[pl.ds(base, b_per_w)], gi_v)
        pltpu.sync_copy(a_hbm.at[pl.ds(base, b_per_w)], ai_v)
        pltpu.sync_copy(o_hbm.at[pl.ds(base, b_per_w)], oi_v)
        pltpu.sync_copy(wg_hbm, tg_v)
        pltpu.sync_copy(wa_hbm, ta_v)
        pltpu.sync_copy(wo_hbm, to_v)

        lane_out = lax.iota(jnp.int32, L) * OUT_D

        @plsc.parallel_loop(0, n_groups, step=1, unroll=4)
        def body(i):
            rowg = gi_v[pl.ds(i * L, L)] << 5
            rowa = ai_v[pl.ds(i * L, L)] << 5
            rowo = oi_v[pl.ds(i * L, L)] << 5
            sbase = lane_out + i * (L * OUT_D)
            for f, (t_v, rowx) in enumerate(
                    ((tg_v, rowg), (ta_v, rowa), (to_v, rowo))):
                for cc in range(D):
                    val = plsc.load_gather(t_v, [rowx + cc])
                    plsc.store_scatter(stage_v, [sbase + (f * D + cc)], val)

        pltpu.sync_copy(stage_v, out_hbm.at[pl.ds(base * OUT_D, b_per_w * OUT_D)])

    out_flat = emb(gender_idx, age_idx, occupation_idx,
                   W_gender.reshape(-1), W_age.reshape(-1),
                   W_occupation.reshape(-1))
    return out_flat.reshape(B, OUT_D)


# speedup vs baseline: 4.1965x; 1.6693x over previous
import functools

import jax
import jax.numpy as jnp
from jax import lax
from jax.experimental import pallas as pl
from jax.experimental.pallas import tpu as pltpu
from jax.experimental.pallas import tpu_sc as plsc

B = 16384
D = 32
L = 16
OUT_D = 3 * D


def _take16(vec, idx):
    return lax.gather(
        vec, idx[:, None],
        lax.GatherDimensionNumbers(offset_dims=(), collapsed_slice_dims=(0,),
                                   start_index_map=(0,)),
        (1,), mode=lax.GatherScatterMode.PROMISE_IN_BOUNDS)


def kernel(gender_idx, age_idx, occupation_idx, W_gender, W_age, W_occupation):
    info = plsc.get_sparse_core_info()
    nw = info.num_cores * info.num_subcores
    b_per_w = B // nw
    n_groups = b_per_w // L
    mesh = plsc.VectorSubcoreMesh(core_axis_name="c", subcore_axis_name="s")

    @functools.partial(
        pl.kernel,
        mesh=mesh,
        out_type=jax.ShapeDtypeStruct((B * OUT_D,), jnp.float32),
        compiler_params=pltpu.CompilerParams(use_tc_tiling_on_sc=False,
                                             needs_layout_passes=False),
        scratch_types=[
            pltpu.VMEM((b_per_w,), jnp.int32),
            pltpu.VMEM((b_per_w,), jnp.int32),
            pltpu.VMEM((b_per_w,), jnp.int32),
            pltpu.VMEM((2 * D,), jnp.float32),
            pltpu.VMEM((7 * D,), jnp.float32),
            pltpu.VMEM((21 * D,), jnp.float32),
            pltpu.VMEM((b_per_w * OUT_D,), jnp.float32),
        ],
    )
    def emb(g_hbm, a_hbm, o_hbm, wg_hbm, wa_hbm, wo_hbm, out_hbm,
            gi_v, ai_v, oi_v, tg_v, ta_v, to_v, stage_v):
        wid = lax.axis_index("s") * info.num_cores + lax.axis_index("c")
        base = wid * b_per_w
        pltpu.sync_copy(g_hbm.at[pl.ds(base, b_per_w)], gi_v)
        pltpu.sync_copy(a_hbm.at[pl.ds(base, b_per_w)], ai_v)
        pltpu.sync_copy(o_hbm.at[pl.ds(base, b_per_w)], oi_v)
        pltpu.sync_copy(wg_hbm, tg_v)
        pltpu.sync_copy(wa_hbm, ta_v)
        pltpu.sync_copy(wo_hbm, to_v)

        lanes16 = lax.iota(jnp.int32, L)

        @plsc.parallel_loop(0, n_groups, step=1, unroll=2)
        def body(i):
            rowg = gi_v[pl.ds(i * L, L)]
            rowa = ai_v[pl.ds(i * L, L)]
            rowo = oi_v[pl.ds(i * L, L)]
            gb = i * (L * OUT_D)
            for l in range(L):
                sel = jnp.full((L,), l, jnp.int32)
                boff = gb + l * OUT_D
                for f, (t_v, rowx) in enumerate(
                        ((tg_v, rowg), (ta_v, rowa), (to_v, rowo))):
                    gaddr = (_take16(rowx, sel) << 5) + lanes16
                    for h in range(2):
                        val = plsc.load_gather(t_v, [gaddr + h * L])
                        stage_v[pl.ds(boff + f * D + h * L, L)] = val

        pltpu.sync_copy(stage_v, out_hbm.at[pl.ds(base * OUT_D, b_per_w * OUT_D)])

    out_flat = emb(gender_idx, age_idx, occupation_idx,
                   W_gender.reshape(-1), W_age.reshape(-1),
                   W_occupation.reshape(-1))
    return out_flat.reshape(B, OUT_D)


# speedup vs baseline: 5.0478x; 1.2029x over previous
import functools

import jax
import jax.numpy as jnp
from jax import lax
from jax.experimental import pallas as pl
from jax.experimental.pallas import tpu as pltpu
from jax.experimental.pallas import tpu_sc as plsc

B = 16384
D = 32
L = 16
OUT_D = 3 * D


def _take16(vec, idx):
    return lax.gather(
        vec, idx[:, None],
        lax.GatherDimensionNumbers(offset_dims=(), collapsed_slice_dims=(0,),
                                   start_index_map=(0,)),
        (1,), mode=lax.GatherScatterMode.PROMISE_IN_BOUNDS)


def kernel(gender_idx, age_idx, occupation_idx, W_gender, W_age, W_occupation):
    info = plsc.get_sparse_core_info()
    nw = info.num_cores * info.num_subcores
    b_per_w = B // nw
    n_groups = b_per_w // L
    mesh = plsc.VectorSubcoreMesh(core_axis_name="c", subcore_axis_name="s")

    @functools.partial(
        pl.kernel,
        mesh=mesh,
        out_type=jax.ShapeDtypeStruct((B, OUT_D), jnp.float32),
        compiler_params=pltpu.CompilerParams(needs_layout_passes=False),
        scratch_types=[
            pltpu.VMEM((b_per_w,), jnp.int32),
            pltpu.VMEM((b_per_w,), jnp.int32),
            pltpu.VMEM((b_per_w,), jnp.int32),
            pltpu.VMEM((2, D), jnp.float32),
            pltpu.VMEM((7, D), jnp.float32),
            pltpu.VMEM((21, D), jnp.float32),
            pltpu.VMEM((b_per_w, OUT_D), jnp.float32),
        ],
    )
    def emb(g_hbm, a_hbm, o_hbm, wg_hbm, wa_hbm, wo_hbm, out_hbm,
            gi_v, ai_v, oi_v, tg_v, ta_v, to_v, stage_v):
        wid = lax.axis_index("s") * info.num_cores + lax.axis_index("c")
        base = wid * b_per_w
        pltpu.sync_copy(g_hbm.at[pl.ds(base, b_per_w)], gi_v)
        pltpu.sync_copy(a_hbm.at[pl.ds(base, b_per_w)], ai_v)
        pltpu.sync_copy(o_hbm.at[pl.ds(base, b_per_w)], oi_v)
        pltpu.sync_copy(wg_hbm, tg_v)
        pltpu.sync_copy(wa_hbm, ta_v)
        pltpu.sync_copy(wo_hbm, to_v)

        lanes16 = lax.iota(jnp.int32, L)

        @plsc.parallel_loop(0, n_groups, step=1, unroll=2)
        def body(i):
            rowg = gi_v[pl.ds(i * L, L)]
            rowa = ai_v[pl.ds(i * L, L)]
            rowo = oi_v[pl.ds(i * L, L)]
            for l in range(L):
                sel = jnp.full((L,), l, jnp.int32)
                bidx = i * L + l
                for f, (t_v, rowx) in enumerate(
                        ((tg_v, rowg), (ta_v, rowa), (to_v, rowo))):
                    rsplat = _take16(rowx, sel)
                    for h in range(2):
                        val = plsc.load_gather(t_v, [rsplat, lanes16 + h * L])
                        stage_v[bidx, pl.ds(f * D + h * L, L)] = val

        pltpu.sync_copy(stage_v, out_hbm.at[pl.ds(base, b_per_w)])

    return emb(gender_idx, age_idx, occupation_idx,
               W_gender, W_age, W_occupation)


# speedup vs baseline: 5.0987x; 1.0101x over previous
import functools

import jax
import jax.numpy as jnp
from jax import lax
from jax.experimental import pallas as pl
from jax.experimental.pallas import tpu as pltpu
from jax.experimental.pallas import tpu_sc as plsc

B = 16384
D = 32
L = 16
OUT_D = 3 * D


def _take16(vec, idx):
    return lax.gather(
        vec, idx[:, None],
        lax.GatherDimensionNumbers(offset_dims=(), collapsed_slice_dims=(0,),
                                   start_index_map=(0,)),
        (1,), mode=lax.GatherScatterMode.PROMISE_IN_BOUNDS)


def kernel(gender_idx, age_idx, occupation_idx, W_gender, W_age, W_occupation):
    info = plsc.get_sparse_core_info()
    nw = info.num_cores * info.num_subcores
    b_per_w = B // nw
    n_groups = b_per_w // L
    mesh = plsc.VectorSubcoreMesh(core_axis_name="c", subcore_axis_name="s")

    @functools.partial(
        pl.kernel,
        mesh=mesh,
        out_type=jax.ShapeDtypeStruct((B, OUT_D), jnp.float32),
        compiler_params=pltpu.CompilerParams(needs_layout_passes=False,
                                             disable_bounds_checks=True,
                                             disable_semaphore_checks=True),
        scratch_types=[
            pltpu.VMEM((b_per_w,), jnp.int32),
            pltpu.VMEM((b_per_w,), jnp.int32),
            pltpu.VMEM((b_per_w,), jnp.int32),
            pltpu.VMEM((2, D), jnp.float32),
            pltpu.VMEM((7, D), jnp.float32),
            pltpu.VMEM((21, D), jnp.float32),
            pltpu.VMEM((b_per_w, OUT_D), jnp.float32),
            pltpu.SemaphoreType.DMA,
            pltpu.SemaphoreType.DMA,
        ],
    )
    def emb(g_hbm, a_hbm, o_hbm, wg_hbm, wa_hbm, wo_hbm, out_hbm,
            gi_v, ai_v, oi_v, tg_v, ta_v, to_v, stage_v, sem_in, sem_out):
        wid = lax.axis_index("s") * info.num_cores + lax.axis_index("c")
        base = wid * b_per_w
        copies = [
            pltpu.async_copy(g_hbm.at[pl.ds(base, b_per_w)], gi_v, sem_in),
            pltpu.async_copy(a_hbm.at[pl.ds(base, b_per_w)], ai_v, sem_in),
            pltpu.async_copy(o_hbm.at[pl.ds(base, b_per_w)], oi_v, sem_in),
            pltpu.async_copy(wg_hbm, tg_v, sem_in),
            pltpu.async_copy(wa_hbm, ta_v, sem_in),
            pltpu.async_copy(wo_hbm, to_v, sem_in),
        ]
        for c in copies:
            c.wait()

        lanes16 = lax.iota(jnp.int32, L)
        n_chunks = 4
        gpc = n_groups // n_chunks
        rows_pc = gpc * L
        out_copies = []
        for chunk in range(n_chunks):

            @plsc.parallel_loop(chunk * gpc, (chunk + 1) * gpc, step=1,
                                unroll=4)
            def body(i):
                rowg = gi_v[pl.ds(i * L, L)]
                rowa = ai_v[pl.ds(i * L, L)]
                rowo = oi_v[pl.ds(i * L, L)]
                for l in range(L):
                    sel = jnp.full((L,), l, jnp.int32)
                    bidx = i * L + l
                    for f, (t_v, rowx) in enumerate(
                            ((tg_v, rowg), (ta_v, rowa), (to_v, rowo))):
                        rsplat = _take16(rowx, sel)
                        for h in range(2):
                            val = plsc.load_gather(
                                t_v, [rsplat, lanes16 + h * L])
                            stage_v[bidx, pl.ds(f * D + h * L, L)] = val

            out_copies.append(pltpu.async_copy(
                stage_v.at[pl.ds(chunk * rows_pc, rows_pc)],
                out_hbm.at[pl.ds(base + chunk * rows_pc, rows_pc)],
                sem_out))
        for c in out_copies:
            c.wait()

    return emb(gender_idx, age_idx, occupation_idx,
               W_gender, W_age, W_occupation)
